# baseline (speedup 1.0000x reference)
import functools

import jax
import jax.numpy as jnp
from jax import lax
from jax.experimental import pallas as pl
from jax.experimental.pallas import tpu as pltpu
from jax.experimental.pallas import tpu_sc as plsc

N_NODES = 10000
NP = 10240
N_EDGES = 320000
CW = 128
CPW = 80
NC, NS = 2, 16
NW = NC * NS
NCHUNK = NW * CPW
E_PAD = NCHUNK * CW
D_IN, D_HID, D_OUT = 128, 64, 16
RPS = NP // NS

G = 4
BPG = 2
NPASS = CPW // (G * BPG)

_sc_mesh = plsc.VectorSubcoreMesh(core_axis_name="c", subcore_axis_name="s")
_sc_params = pltpu.CompilerParams(use_tc_tiling_on_sc=False,
                                  needs_layout_passes=False)


@functools.partial(
    pl.kernel,
    out_type=jax.ShapeDtypeStruct((NW, NP), jnp.float32),
    mesh=_sc_mesh,
    compiler_params=_sc_params,
    scratch_types=[
        pltpu.VMEM((CPW, CW), jnp.int32),
        pltpu.VMEM((NP,), jnp.float32),
    ],
)
def _deg_kernel(col2_hbm, zeros_hbm, out_hbm, cidx, hist):
    c = lax.axis_index("c")
    s = lax.axis_index("s")
    w = s * NC + c
    pltpu.sync_copy(zeros_hbm, hist)
    pltpu.sync_copy(col2_hbm.at[pl.ds(w * CPW, CPW)], cidx)
    ones = jnp.full((16,), 1.0, jnp.float32)

    def body(j, carry):
        for k in range(CW // 16):
            idx = cidx[j, pl.ds(k * 16, 16)]
            plsc.addupdate_scatter(hist, [idx], ones)
        return carry

    lax.fori_loop(0, CPW, body, 0)
    pltpu.sync_copy(hist, out_hbm.at[w])


def _make_agg(D):
    @functools.partial(
        pl.kernel,
        out_type=jax.ShapeDtypeStruct((NC, NP, D), jnp.float32),
        mesh=_sc_mesh,
        compiler_params=_sc_params,
        scratch_types=[
            pltpu.VMEM((CPW, CW), jnp.int32),
            pltpu.VMEM((CPW, CW), jnp.int32),
            pltpu.VMEM((G * BPG * CW, D), jnp.float32),
            pltpu.VMEM_SHARED((NP, D), jnp.float32),
            [pltpu.SemaphoreType.DMA] * G,
            [pltpu.SemaphoreType.DMA] * G,
        ],
    )
    def agg(g_hbm, row2_hbm, col2_hbm, zeros_hbm, out_hbm,
            ridx, cidx, gbuf, acc, gsems, ssems):
        c = lax.axis_index("c")
        s = lax.axis_index("s")
        w = s * NC + c
        pltpu.sync_copy(zeros_hbm.at[pl.ds(s * RPS, RPS)],
                        acc.at[pl.ds(s * RPS, RPS)])
        pltpu.sync_copy(row2_hbm.at[pl.ds(w * CPW, CPW)], ridx)
        pltpu.sync_copy(col2_hbm.at[pl.ds(w * CPW, CPW)], cidx)
        plsc.subcore_barrier()

        def gather(gi, j, k):
            pltpu.async_copy(g_hbm.at[ridx.at[k]],
                             gbuf.at[pl.ds((gi * BPG + j) * CW, CW)],
                             gsems[gi])

        def scatter(gi, j, k):
            pltpu.async_copy(gbuf.at[pl.ds((gi * BPG + j) * CW, CW)],
                             acc.at[cidx.at[k]], ssems[gi], add=True)

        def drain(sem, gi):
            pltpu.make_async_copy(
                zeros_hbm.at[pl.ds(0, BPG * CW)],
                gbuf.at[pl.ds(gi * BPG * CW, BPG * CW)], sem).wait()

        for gi in range(G):
            for j in range(BPG):
                gather(gi, j, gi * BPG + j)

        def body(t, carry):
            for gi in range(G):
                rp = G * (t - 1) + gi
                drain(gsems[gi], gi)
                for j in range(BPG):
                    scatter(gi, j, rp * BPG + j)
                drain(ssems[gi], gi)
                rc = G * t + gi
                for j in range(BPG):
                    gather(gi, j, rc * BPG + j)
            return carry

        lax.fori_loop(1, NPASS, body, 0)

        for gi in range(G):
            rp = G * (NPASS - 1) + gi
            drain(gsems[gi], gi)
            for j in range(BPG):
                scatter(gi, j, rp * BPG + j)
            drain(ssems[gi], gi)

        plsc.subcore_barrier()
        pltpu.sync_copy(acc.at[pl.ds(s * RPS, RPS)],
                        out_hbm.at[c, pl.ds(s * RPS, RPS)])

    return agg


_agg64 = _make_agg(D_HID)
_agg16 = _make_agg(D_OUT)


BR = 1024
GRID = NP // BR


def _dinv_of(deg_ref):
    i = pl.program_id(0)
    sl = deg_ref[:, pl.ds(i * BR, BR)]
    return lax.rsqrt(jnp.sum(sl, axis=0) + 1.0)[:, None]


def _mm_body(x_ref, w1_ref, h1_ref):
    h1_ref[...] = jnp.dot(x_ref[...], w1_ref[...],
                          preferred_element_type=jnp.float32)


def _scale_body(deg_ref, h_ref, g_ref):
    g_ref[...] = h_ref[...] * _dinv_of(deg_ref)


def _post1_body(deg_ref, s_ref, g1_ref, b1_ref, w2_ref, g2_ref):
    dinv = _dinv_of(deg_ref)
    out1 = jnp.maximum((s_ref[0] + s_ref[1] + g1_ref[...]) * dinv + b1_ref[...],
                       0.0)
    h2 = jnp.dot(out1, w2_ref[...], preferred_element_type=jnp.float32)
    g2_ref[...] = h2 * dinv


def _post2_body(deg_ref, t_ref, g2_ref, b2_ref, o_ref):
    dinv = _dinv_of(deg_ref)
    o = (t_ref[0] + t_ref[1] + g2_ref[...]) * dinv + b2_ref[...]
    m = jnp.max(o, axis=1, keepdims=True)
    o_ref[...] = o - (jnp.log(jnp.sum(jnp.exp(o - m), axis=1, keepdims=True)) + m)


def _deg_spec():
    return pl.BlockSpec((NW, NP), lambda i: (0, 0))


def _mm(x_p, W1):
    return pl.pallas_call(
        _mm_body,
        grid=(GRID,),
        in_specs=[
            pl.BlockSpec((BR, D_IN), lambda i: (i, 0)),
            pl.BlockSpec((D_IN, D_HID), lambda i: (0, 0)),
        ],
        out_specs=pl.BlockSpec((BR, D_HID), lambda i: (i, 0)),
        out_shape=jax.ShapeDtypeStruct((NP, D_HID), jnp.float32),
    )(x_p, W1)


def _scale(degP, h1):
    return pl.pallas_call(
        _scale_body,
        grid=(GRID,),
        in_specs=[
            _deg_spec(),
            pl.BlockSpec((BR, D_HID), lambda i: (i, 0)),
        ],
        out_specs=pl.BlockSpec((BR, D_HID), lambda i: (i, 0)),
        out_shape=jax.ShapeDtypeStruct((NP, D_HID), jnp.float32),
    )(degP, h1)


def _post1(degP, S, g1, b1, W2):
    return pl.pallas_call(
        _post1_body,
        grid=(GRID,),
        in_specs=[
            _deg_spec(),
            pl.BlockSpec((NC, BR, D_HID), lambda i: (0, i, 0)),
            pl.BlockSpec((BR, D_HID), lambda i: (i, 0)),
            pl.BlockSpec((1, D_HID), lambda i: (0, 0)),
            pl.BlockSpec((D_HID, D_OUT), lambda i: (0, 0)),
        ],
        out_specs=pl.BlockSpec((BR, D_OUT), lambda i: (i, 0)),
        out_shape=jax.ShapeDtypeStruct((NP, D_OUT), jnp.float32),
    )(degP, S, g1, b1, W2)


def _post2(degP, T, g2, b2):
    return pl.pallas_call(
        _post2_body,
        grid=(GRID,),
        in_specs=[
            _deg_spec(),
            pl.BlockSpec((NC, BR, D_OUT), lambda i: (0, i, 0)),
            pl.BlockSpec((BR, D_OUT), lambda i: (i, 0)),
            pl.BlockSpec((1, D_OUT), lambda i: (0, 0)),
        ],
        out_specs=pl.BlockSpec((BR, D_OUT), lambda i: (i, 0)),
        out_shape=jax.ShapeDtypeStruct((NP, D_OUT), jnp.float32),
    )(degP, T, g2, b2)


def kernel(x, edge_index, W1, b1, W2, b2):
    ei = edge_index.astype(jnp.int32)
    padidx = N_NODES + (jnp.arange(E_PAD - N_EDGES, dtype=jnp.int32)
                        % (NP - N_NODES))
    row2 = jnp.concatenate([ei[0], padidx]).reshape(NCHUNK, CW)
    col2 = jnp.concatenate([ei[1], padidx]).reshape(NCHUNK, CW)
    z1 = jnp.zeros((NP,), jnp.float32)
    z64 = jnp.zeros((NP, D_HID), jnp.float32)
    z16 = jnp.zeros((NP, D_OUT), jnp.float32)

    x_p = jnp.pad(x, ((0, NP - N_NODES), (0, 0)))
    degP = _deg_kernel(col2, z1)
    h1 = _mm(x_p, W1)
    g1 = _scale(degP, h1)
    S = _agg64(g1, row2, col2, z64)
    g2 = _post1(degP, S, g1, b1.reshape(1, -1), W2)
    T = _agg16(g2, row2, col2, z16)
    o = _post2(degP, T, g2, b2.reshape(1, -1))
    return o[:N_NODES]


# speedup vs baseline: 1.0036x; 1.0036x over previous
import functools

import jax
import jax.numpy as jnp
from jax import lax
from jax.experimental import pallas as pl
from jax.experimental.pallas import tpu as pltpu
from jax.experimental.pallas import tpu_sc as plsc

N_NODES = 10000
NP = 10240
N_EDGES = 320000
CW = 128
CPW = 80
NC, NS = 2, 16
NW = NC * NS
NCHUNK = NW * CPW
E_PAD = NCHUNK * CW
D_IN, D_HID, D_OUT = 128, 64, 16
RPS = NP // NS

BPG = 2

_sc_mesh = plsc.VectorSubcoreMesh(core_axis_name="c", subcore_axis_name="s")
_sc_params = pltpu.CompilerParams(use_tc_tiling_on_sc=False,
                                  needs_layout_passes=False)
_deg_params = pltpu.CompilerParams(needs_layout_passes=False)


@functools.partial(
    pl.kernel,
    out_type=jax.ShapeDtypeStruct((NW, NP), jnp.float32),
    mesh=_sc_mesh,
    compiler_params=_deg_params,
    scratch_types=[
        pltpu.VMEM((CPW, CW), jnp.int32),
        pltpu.VMEM((NP,), jnp.float32),
    ],
)
def _deg_kernel(col2_hbm, zeros_hbm, out_hbm, cidx, hist):
    c = lax.axis_index("c")
    s = lax.axis_index("s")
    w = s * NC + c
    pltpu.sync_copy(zeros_hbm, hist)
    pltpu.sync_copy(col2_hbm.at[pl.ds(w * CPW, CPW)], cidx)
    ones = jnp.full((16,), 1.0, jnp.float32)

    def body(j, carry):
        for k in range(CW // 16):
            idx = cidx[j, pl.ds(k * 16, 16)]
            plsc.addupdate_scatter(hist, [idx], ones)
        return carry

    lax.fori_loop(0, CPW, body, 0)
    pltpu.sync_copy(hist, out_hbm.at[w])


def _make_agg(D, G):
    NPASS = CPW // (G * BPG)
    @functools.partial(
        pl.kernel,
        out_type=jax.ShapeDtypeStruct((NC, NP, D), jnp.float32),
        mesh=_sc_mesh,
        compiler_params=_sc_params,
        scratch_types=[
            pltpu.VMEM((CPW, CW), jnp.int32),
            pltpu.VMEM((CPW, CW), jnp.int32),
            pltpu.VMEM((G * BPG * CW, D), jnp.float32),
            pltpu.VMEM_SHARED((NP, D), jnp.float32),
            [pltpu.SemaphoreType.DMA] * G,
            [pltpu.SemaphoreType.DMA] * G,
        ],
    )
    def agg(g_hbm, row2_hbm, col2_hbm, zeros_hbm, out_hbm,
            ridx, cidx, gbuf, acc, gsems, ssems):
        c = lax.axis_index("c")
        s = lax.axis_index("s")
        w = s * NC + c
        pltpu.sync_copy(zeros_hbm.at[pl.ds(s * RPS, RPS)],
                        acc.at[pl.ds(s * RPS, RPS)])
        pltpu.sync_copy(row2_hbm.at[pl.ds(w * CPW, CPW)], ridx)
        pltpu.sync_copy(col2_hbm.at[pl.ds(w * CPW, CPW)], cidx)
        plsc.subcore_barrier()

        def gather(gi, j, k):
            pltpu.async_copy(g_hbm.at[ridx.at[k]],
                             gbuf.at[pl.ds((gi * BPG + j) * CW, CW)],
                             gsems[gi])

        def scatter(gi, j, k):
            pltpu.async_copy(gbuf.at[pl.ds((gi * BPG + j) * CW, CW)],
                             acc.at[cidx.at[k]], ssems[gi], add=True)

        def drain(sem, gi):
            pltpu.make_async_copy(
                zeros_hbm.at[pl.ds(0, BPG * CW)],
                gbuf.at[pl.ds(gi * BPG * CW, BPG * CW)], sem).wait()

        for gi in range(G):
            for j in range(BPG):
                gather(gi, j, gi * BPG + j)

        def body(t, carry):
            for gi in range(G):
                rp = G * (t - 1) + gi
                drain(gsems[gi], gi)
                for j in range(BPG):
                    scatter(gi, j, rp * BPG + j)
                drain(ssems[gi], gi)
                rc = G * t + gi
                for j in range(BPG):
                    gather(gi, j, rc * BPG + j)
            return carry

        lax.fori_loop(1, NPASS, body, 0)

        for gi in range(G):
            rp = G * (NPASS - 1) + gi
            drain(gsems[gi], gi)
            for j in range(BPG):
                scatter(gi, j, rp * BPG + j)
            drain(ssems[gi], gi)

        plsc.subcore_barrier()
        pltpu.sync_copy(acc.at[pl.ds(s * RPS, RPS)],
                        out_hbm.at[c, pl.ds(s * RPS, RPS)])

    return agg


_agg64 = _make_agg(D_HID, 4)
_agg16 = _make_agg(D_OUT, 8)


BR = 1024
GRID = NP // BR


def _dinv_of(deg_ref):
    i = pl.program_id(0)
    sl = deg_ref[:, pl.ds(i * BR, BR)]
    return lax.rsqrt(jnp.sum(sl, axis=0) + 1.0)[:, None]


def _mm_body(x_ref, w1_ref, h1_ref):
    h1_ref[...] = jnp.dot(x_ref[...], w1_ref[...],
                          preferred_element_type=jnp.float32)


def _scale_body(deg_ref, h_ref, g_ref):
    g_ref[...] = h_ref[...] * _dinv_of(deg_ref)


def _post1_body(deg_ref, s_ref, g1_ref, b1_ref, w2_ref, g2_ref):
    dinv = _dinv_of(deg_ref)
    out1 = jnp.maximum((s_ref[0] + s_ref[1] + g1_ref[...]) * dinv + b1_ref[...],
                       0.0)
    h2 = jnp.dot(out1, w2_ref[...], preferred_element_type=jnp.float32)
    g2_ref[...] = h2 * dinv


def _post2_body(deg_ref, t_ref, g2_ref, b2_ref, o_ref):
    dinv = _dinv_of(deg_ref)
    o = (t_ref[0] + t_ref[1] + g2_ref[...]) * dinv + b2_ref[...]
    m = jnp.max(o, axis=1, keepdims=True)
    o_ref[...] = o - (jnp.log(jnp.sum(jnp.exp(o - m), axis=1, keepdims=True)) + m)


def _deg_spec():
    return pl.BlockSpec((NW, NP), lambda i: (0, 0))


def _mm(x_p, W1):
    return pl.pallas_call(
        _mm_body,
        grid=(GRID,),
        in_specs=[
            pl.BlockSpec((BR, D_IN), lambda i: (i, 0)),
            pl.BlockSpec((D_IN, D_HID), lambda i: (0, 0)),
        ],
        out_specs=pl.BlockSpec((BR, D_HID), lambda i: (i, 0)),
        out_shape=jax.ShapeDtypeStruct((NP, D_HID), jnp.float32),
    )(x_p, W1)


def _scale(degP, h1):
    return pl.pallas_call(
        _scale_body,
        grid=(GRID,),
        in_specs=[
            _deg_spec(),
            pl.BlockSpec((BR, D_HID), lambda i: (i, 0)),
        ],
        out_specs=pl.BlockSpec((BR, D_HID), lambda i: (i, 0)),
        out_shape=jax.ShapeDtypeStruct((NP, D_HID), jnp.float32),
    )(degP, h1)


def _post1(degP, S, g1, b1, W2):
    return pl.pallas_call(
        _post1_body,
        grid=(GRID,),
        in_specs=[
            _deg_spec(),
            pl.BlockSpec((NC, BR, D_HID), lambda i: (0, i, 0)),
            pl.BlockSpec((BR, D_HID), lambda i: (i, 0)),
            pl.BlockSpec((1, D_HID), lambda i: (0, 0)),
            pl.BlockSpec((D_HID, D_OUT), lambda i: (0, 0)),
        ],
        out_specs=pl.BlockSpec((BR, D_OUT), lambda i: (i, 0)),
        out_shape=jax.ShapeDtypeStruct((NP, D_OUT), jnp.float32),
    )(degP, S, g1, b1, W2)


def _post2(degP, T, g2, b2):
    return pl.pallas_call(
        _post2_body,
        grid=(GRID,),
        in_specs=[
            _deg_spec(),
            pl.BlockSpec((NC, BR, D_OUT), lambda i: (0, i, 0)),
            pl.BlockSpec((BR, D_OUT), lambda i: (i, 0)),
            pl.BlockSpec((1, D_OUT), lambda i: (0, 0)),
        ],
        out_specs=pl.BlockSpec((BR, D_OUT), lambda i: (i, 0)),
        out_shape=jax.ShapeDtypeStruct((NP, D_OUT), jnp.float32),
    )(degP, T, g2, b2)


def kernel(x, edge_index, W1, b1, W2, b2):
    ei = edge_index.astype(jnp.int32)
    padidx = N_NODES + (jnp.arange(E_PAD - N_EDGES, dtype=jnp.int32)
                        % (NP - N_NODES))
    row2 = jnp.concatenate([ei[0], padidx]).reshape(NCHUNK, CW)
    col2 = jnp.concatenate([ei[1], padidx]).reshape(NCHUNK, CW)
    z1 = jnp.zeros((NP,), jnp.float32)
    z64 = jnp.zeros((NP, D_HID), jnp.float32)
    z16 = jnp.zeros((NP, D_OUT), jnp.float32)

    x_p = jnp.pad(x, ((0, NP - N_NODES), (0, 0)))
    degP = _deg_kernel(col2, z1)
    h1 = _mm(x_p, W1)
    g1 = _scale(degP, h1)
    S = _agg64(g1, row2, col2, z64)
    g2 = _post1(degP, S, g1, b1.reshape(1, -1), W2)
    T = _agg16(g2, row2, col2, z16)
    o = _post2(degP, T, g2, b2.reshape(1, -1))
    return o[:N_NODES]


# speedup vs baseline: 1.0588x; 1.0550x over previous
import functools

import jax
import jax.numpy as jnp
from jax import lax
from jax.experimental import pallas as pl
from jax.experimental.pallas import tpu as pltpu
from jax.experimental.pallas import tpu_sc as plsc

N_NODES = 10000
NP = 10240
N_EDGES = 320000
CW = 128
CPW = 80
NC, NS = 2, 16
NW = NC * NS
NCHUNK = NW * CPW
E_PAD = NCHUNK * CW
D_IN, D_HID, D_OUT = 128, 64, 16
RPS = NP // NS

BPG = 2

NCHUNK_R = N_EDGES // CW
RW_LAST = NCHUNK_R - (NW - 1) * CPW
PW_LAST = CPW - RW_LAST

_sc_mesh = plsc.VectorSubcoreMesh(core_axis_name="c", subcore_axis_name="s")
_sc_params = pltpu.CompilerParams(use_tc_tiling_on_sc=False,
                                  needs_layout_passes=False)


def _hoist_idx(ei3_hbm, padc_hbm, idx_v, which, w):
    @pl.when(w < NW - 1)
    def _():
        pltpu.sync_copy(ei3_hbm.at[which, pl.ds(w * CPW, CPW)], idx_v)

    @pl.when(w == NW - 1)
    def _():
        pltpu.sync_copy(ei3_hbm.at[which, pl.ds((NW - 1) * CPW, RW_LAST)],
                        idx_v.at[pl.ds(0, RW_LAST)])
        pltpu.sync_copy(padc_hbm, idx_v.at[pl.ds(RW_LAST, PW_LAST)])


@functools.partial(
    pl.kernel,
    out_type=jax.ShapeDtypeStruct((NW, NP), jnp.float32),
    mesh=_sc_mesh,
    compiler_params=_sc_params,
    scratch_types=[
        pltpu.VMEM((CPW, CW), jnp.int32),
        pltpu.VMEM((NP,), jnp.float32),
    ],
)
def _deg_kernel(ei3_hbm, padc_hbm, zeros_hbm, out_hbm, cidx, hist):
    c = lax.axis_index("c")
    s = lax.axis_index("s")
    w = s * NC + c
    pltpu.sync_copy(zeros_hbm, hist)
    _hoist_idx(ei3_hbm, padc_hbm, cidx, 1, w)
    ones = jnp.full((16,), 1.0, jnp.float32)

    def body(j, carry):
        for k in range(CW // 16):
            idx = cidx[j, pl.ds(k * 16, 16)]
            plsc.addupdate_scatter(hist, [idx], ones)
        return carry

    lax.fori_loop(0, CPW, body, 0)
    pltpu.sync_copy(hist, out_hbm.at[w])


def _make_agg(D, G):
    NPASS = CPW // (G * BPG)
    @functools.partial(
        pl.kernel,
        out_type=jax.ShapeDtypeStruct((NC, NP, D), jnp.float32),
        mesh=_sc_mesh,
        compiler_params=_sc_params,
        scratch_types=[
            pltpu.VMEM((CPW, CW), jnp.int32),
            pltpu.VMEM((CPW, CW), jnp.int32),
            pltpu.VMEM((G * BPG * CW, D), jnp.float32),
            pltpu.VMEM_SHARED((NP, D), jnp.float32),
            [pltpu.SemaphoreType.DMA] * G,
            [pltpu.SemaphoreType.DMA] * G,
        ],
    )
    def agg(g_hbm, ei3_hbm, padc_hbm, zeros_hbm, out_hbm,
            ridx, cidx, gbuf, acc, gsems, ssems):
        c = lax.axis_index("c")
        s = lax.axis_index("s")
        w = s * NC + c
        pltpu.sync_copy(zeros_hbm.at[pl.ds(s * RPS, RPS)],
                        acc.at[pl.ds(s * RPS, RPS)])
        _hoist_idx(ei3_hbm, padc_hbm, ridx, 0, w)
        _hoist_idx(ei3_hbm, padc_hbm, cidx, 1, w)
        plsc.subcore_barrier()

        def gather(gi, j, k):
            pltpu.async_copy(g_hbm.at[ridx.at[k]],
                             gbuf.at[pl.ds((gi * BPG + j) * CW, CW)],
                             gsems[gi])

        def scatter(gi, j, k):
            pltpu.async_copy(gbuf.at[pl.ds((gi * BPG + j) * CW, CW)],
                             acc.at[cidx.at[k]], ssems[gi], add=True)

        def drain(sem, gi):
            pltpu.make_async_copy(
                zeros_hbm.at[pl.ds(0, BPG * CW)],
                gbuf.at[pl.ds(gi * BPG * CW, BPG * CW)], sem).wait()

        for gi in range(G):
            for j in range(BPG):
                gather(gi, j, gi * BPG + j)

        def body(t, carry):
            for gi in range(G):
                rp = G * (t - 1) + gi
                drain(gsems[gi], gi)
                for j in range(BPG):
                    scatter(gi, j, rp * BPG + j)
                drain(ssems[gi], gi)
                rc = G * t + gi
                for j in range(BPG):
                    gather(gi, j, rc * BPG + j)
            return carry

        lax.fori_loop(1, NPASS, body, 0)

        for gi in range(G):
            rp = G * (NPASS - 1) + gi
            drain(gsems[gi], gi)
            for j in range(BPG):
                scatter(gi, j, rp * BPG + j)
            drain(ssems[gi], gi)

        plsc.subcore_barrier()
        pltpu.sync_copy(acc.at[pl.ds(s * RPS, RPS)],
                        out_hbm.at[c, pl.ds(s * RPS, RPS)])

    return agg


_agg64 = _make_agg(D_HID, 4)
_agg16 = _make_agg(D_OUT, 8)


BR = 1024
GRID = NP // BR


def _dinv_of(deg_ref):
    i = pl.program_id(0)
    sl = deg_ref[:, pl.ds(i * BR, BR)]
    return lax.rsqrt(jnp.sum(sl, axis=0) + 1.0)[:, None]


def _mm_body(x_ref, w1_ref, h1_ref):
    h1_ref[...] = jnp.dot(x_ref[...], w1_ref[...],
                          preferred_element_type=jnp.float32)


def _scale_body(deg_ref, h_ref, g_ref):
    g_ref[...] = h_ref[...] * _dinv_of(deg_ref)


def _post1_body(deg_ref, s_ref, g1_ref, b1_ref, w2_ref, g2_ref):
    dinv = _dinv_of(deg_ref)
    out1 = jnp.maximum((s_ref[0] + s_ref[1] + g1_ref[...]) * dinv + b1_ref[...],
                       0.0)
    h2 = jnp.dot(out1, w2_ref[...], preferred_element_type=jnp.float32)
    g2_ref[...] = h2 * dinv


def _post2_body(deg_ref, t_ref, g2_ref, b2_ref, o_ref):
    dinv = _dinv_of(deg_ref)
    o = (t_ref[0] + t_ref[1] + g2_ref[...]) * dinv + b2_ref[...]
    m = jnp.max(o, axis=1, keepdims=True)
    o_ref[...] = o - (jnp.log(jnp.sum(jnp.exp(o - m), axis=1, keepdims=True)) + m)


def _deg_spec():
    return pl.BlockSpec((NW, NP), lambda i: (0, 0))


def _mm(x_p, W1):
    return pl.pallas_call(
        _mm_body,
        grid=(GRID,),
        in_specs=[
            pl.BlockSpec((BR, D_IN), lambda i: (i, 0)),
            pl.BlockSpec((D_IN, D_HID), lambda i: (0, 0)),
        ],
        out_specs=pl.BlockSpec((BR, D_HID), lambda i: (i, 0)),
        out_shape=jax.ShapeDtypeStruct((NP, D_HID), jnp.float32),
    )(x_p, W1)


def _scale(degP, h1):
    return pl.pallas_call(
        _scale_body,
        grid=(GRID,),
        in_specs=[
            _deg_spec(),
            pl.BlockSpec((BR, D_HID), lambda i: (i, 0)),
        ],
        out_specs=pl.BlockSpec((BR, D_HID), lambda i: (i, 0)),
        out_shape=jax.ShapeDtypeStruct((NP, D_HID), jnp.float32),
    )(degP, h1)


def _post1(degP, S, g1, b1, W2):
    return pl.pallas_call(
        _post1_body,
        grid=(GRID,),
        in_specs=[
            _deg_spec(),
            pl.BlockSpec((NC, BR, D_HID), lambda i: (0, i, 0)),
            pl.BlockSpec((BR, D_HID), lambda i: (i, 0)),
            pl.BlockSpec((1, D_HID), lambda i: (0, 0)),
            pl.BlockSpec((D_HID, D_OUT), lambda i: (0, 0)),
        ],
        out_specs=pl.BlockSpec((BR, D_OUT), lambda i: (i, 0)),
        out_shape=jax.ShapeDtypeStruct((NP, D_OUT), jnp.float32),
    )(degP, S, g1, b1, W2)


def _post2(degP, T, g2, b2):
    return pl.pallas_call(
        _post2_body,
        grid=(GRID,),
        in_specs=[
            _deg_spec(),
            pl.BlockSpec((NC, BR, D_OUT), lambda i: (0, i, 0)),
            pl.BlockSpec((BR, D_OUT), lambda i: (i, 0)),
            pl.BlockSpec((1, D_OUT), lambda i: (0, 0)),
        ],
        out_specs=pl.BlockSpec((BR, D_OUT), lambda i: (i, 0)),
        out_shape=jax.ShapeDtypeStruct((NP, D_OUT), jnp.float32),
    )(degP, T, g2, b2)


def kernel(x, edge_index, W1, b1, W2, b2):
    ei3 = edge_index.astype(jnp.int32).reshape(2, NCHUNK_R, CW)
    padc = (N_NODES + (jnp.arange(PW_LAST * CW, dtype=jnp.int32)
                       % (NP - N_NODES))).reshape(PW_LAST, CW)
    x_p = jnp.pad(x, ((0, NP - N_NODES), (0, 0)))
    z1 = jnp.zeros((NP,), jnp.float32)
    z64 = jnp.zeros((NP, D_HID), jnp.float32)
    z16 = jnp.zeros((NP, D_OUT), jnp.float32)

    degP = _deg_kernel(ei3, padc, z1)
    h1 = _mm(x_p, W1)
    g1 = _scale(degP, h1)
    S = _agg64(g1, ei3, padc, z64)
    g2 = _post1(degP, S, g1, b1.reshape(1, -1), W2)
    T = _agg16(g2, ei3, padc, z16)
    o = _post2(degP, T, g2, b2.reshape(1, -1))
    return o[:N_NODES]


# speedup vs baseline: 1.1055x; 1.0441x over previous
import functools

import jax
import jax.numpy as jnp
from jax import lax
from jax.experimental import pallas as pl
from jax.experimental.pallas import tpu as pltpu
from jax.experimental.pallas import tpu_sc as plsc

N_NODES = 10000
NP = 10240
N_EDGES = 320000
CW = 128
CPW = 80
NC, NS = 2, 16
NW = NC * NS
NCHUNK = NW * CPW
E_PAD = NCHUNK * CW
D_IN, D_HID, D_OUT = 128, 64, 16
RPS = NP // NS

BPG = 2

NCHUNK_R = N_EDGES // CW
RW_LAST = NCHUNK_R - (NW - 1) * CPW
PW_LAST = CPW - RW_LAST

_sc_mesh = plsc.VectorSubcoreMesh(core_axis_name="c", subcore_axis_name="s")
_sc_params = pltpu.CompilerParams(use_tc_tiling_on_sc=False,
                                  needs_layout_passes=False)


def _hoist_idx(ei3_hbm, padc_hbm, idx_v, which, w):
    @pl.when(w < NW - 1)
    def _():
        pltpu.sync_copy(ei3_hbm.at[which, pl.ds(w * CPW, CPW)], idx_v)

    @pl.when(w == NW - 1)
    def _():
        pltpu.sync_copy(ei3_hbm.at[which, pl.ds((NW - 1) * CPW, RW_LAST)],
                        idx_v.at[pl.ds(0, RW_LAST)])
        pltpu.sync_copy(padc_hbm, idx_v.at[pl.ds(RW_LAST, PW_LAST)])


@functools.partial(
    pl.kernel,
    out_type=jax.ShapeDtypeStruct((NW, NP), jnp.float32),
    mesh=_sc_mesh,
    compiler_params=_sc_params,
    scratch_types=[
        pltpu.VMEM((CPW, CW), jnp.int32),
        pltpu.VMEM((NP,), jnp.float32),
    ],
)
def _deg_kernel(ei3_hbm, padc_hbm, zeros_hbm, out_hbm, cidx, hist):
    c = lax.axis_index("c")
    s = lax.axis_index("s")
    w = s * NC + c
    pltpu.sync_copy(zeros_hbm, hist)
    _hoist_idx(ei3_hbm, padc_hbm, cidx, 1, w)
    ones = jnp.full((16,), 1.0, jnp.float32)

    def body(j, carry):
        for k in range(CW // 16):
            idx = cidx[j, pl.ds(k * 16, 16)]
            plsc.addupdate_scatter(hist, [idx], ones)
        return carry

    lax.fori_loop(0, CPW, body, 0)
    pltpu.sync_copy(hist, out_hbm.at[w])


def _make_agg(D, G):
    NPASS = CPW // (G * BPG)
    @functools.partial(
        pl.kernel,
        out_type=jax.ShapeDtypeStruct((NC, NP, D), jnp.float32),
        mesh=_sc_mesh,
        compiler_params=_sc_params,
        scratch_types=[
            pltpu.VMEM((CPW, CW), jnp.int32),
            pltpu.VMEM((CPW, CW), jnp.int32),
            pltpu.VMEM((G * BPG * CW, D), jnp.float32),
            pltpu.VMEM_SHARED((NP, D), jnp.float32),
            [pltpu.SemaphoreType.DMA] * G,
            [pltpu.SemaphoreType.DMA] * G,
        ],
    )
    def agg(g_hbm, ei3_hbm, padc_hbm, zeros_hbm, out_hbm,
            ridx, cidx, gbuf, acc, gsems, ssems):
        c = lax.axis_index("c")
        s = lax.axis_index("s")
        w = s * NC + c
        pltpu.sync_copy(zeros_hbm.at[pl.ds(s * RPS, RPS)],
                        acc.at[pl.ds(s * RPS, RPS)])
        _hoist_idx(ei3_hbm, padc_hbm, ridx, 0, w)
        _hoist_idx(ei3_hbm, padc_hbm, cidx, 1, w)
        plsc.subcore_barrier()

        def gather(gi, j, k):
            pltpu.async_copy(g_hbm.at[ridx.at[k]],
                             gbuf.at[pl.ds((gi * BPG + j) * CW, CW)],
                             gsems[gi])

        def scatter(gi, j, k):
            pltpu.async_copy(gbuf.at[pl.ds((gi * BPG + j) * CW, CW)],
                             acc.at[cidx.at[k]], ssems[gi], add=True)

        def drain(sem, gi):
            pltpu.make_async_copy(
                zeros_hbm.at[pl.ds(0, BPG * CW)],
                gbuf.at[pl.ds(gi * BPG * CW, BPG * CW)], sem).wait()

        for gi in range(G):
            for j in range(BPG):
                gather(gi, j, gi * BPG + j)

        def body(t, carry):
            for gi in range(G):
                rp = G * (t - 1) + gi
                drain(gsems[gi], gi)
                for j in range(BPG):
                    scatter(gi, j, rp * BPG + j)
                drain(ssems[gi], gi)
                rc = G * t + gi
                for j in range(BPG):
                    gather(gi, j, rc * BPG + j)
            return carry

        lax.fori_loop(1, NPASS, body, 0)

        for gi in range(G):
            rp = G * (NPASS - 1) + gi
            drain(gsems[gi], gi)
            for j in range(BPG):
                scatter(gi, j, rp * BPG + j)
            drain(ssems[gi], gi)

        plsc.subcore_barrier()
        pltpu.sync_copy(acc.at[pl.ds(s * RPS, RPS)],
                        out_hbm.at[c, pl.ds(s * RPS, RPS)])

    return agg


_agg64 = _make_agg(D_HID, 4)
_agg16 = _make_agg(D_OUT, 8)


BR = 2048
GRID = NP // BR


def _dinv_of(deg_ref):
    i = pl.program_id(0)
    sl = deg_ref[:, pl.ds(i * BR, BR)]
    return lax.rsqrt(jnp.sum(sl, axis=0) + 1.0)[:, None]


def _mm_body(x_ref, w1_ref, h1_ref):
    h1_ref[...] = jnp.dot(x_ref[...], w1_ref[...],
                          preferred_element_type=jnp.float32)


def _scale_body(deg_ref, h_ref, g_ref):
    g_ref[...] = h_ref[...] * _dinv_of(deg_ref)


def _post1_body(deg_ref, s_ref, g1_ref, b1_ref, w2_ref, g2_ref):
    dinv = _dinv_of(deg_ref)
    out1 = jnp.maximum((s_ref[0] + s_ref[1] + g1_ref[...]) * dinv + b1_ref[...],
                       0.0)
    h2 = jnp.dot(out1, w2_ref[...], preferred_element_type=jnp.float32)
    g2_ref[...] = h2 * dinv


def _post2_body(deg_ref, t_ref, g2_ref, b2_ref, o_ref):
    dinv = _dinv_of(deg_ref)
    o = (t_ref[0] + t_ref[1] + g2_ref[...]) * dinv + b2_ref[...]
    m = jnp.max(o, axis=1, keepdims=True)
    o_ref[...] = o - (jnp.log(jnp.sum(jnp.exp(o - m), axis=1, keepdims=True)) + m)


def _deg_spec():
    return pl.BlockSpec((NW, NP), lambda i: (0, 0))


def _mm(x_p, W1):
    return pl.pallas_call(
        _mm_body,
        grid=(GRID,),
        in_specs=[
            pl.BlockSpec((BR, D_IN), lambda i: (i, 0)),
            pl.BlockSpec((D_IN, D_HID), lambda i: (0, 0)),
        ],
        out_specs=pl.BlockSpec((BR, D_HID), lambda i: (i, 0)),
        out_shape=jax.ShapeDtypeStruct((NP, D_HID), jnp.float32),
    )(x_p, W1)


def _scale(degP, h1):
    return pl.pallas_call(
        _scale_body,
        grid=(GRID,),
        in_specs=[
            _deg_spec(),
            pl.BlockSpec((BR, D_HID), lambda i: (i, 0)),
        ],
        out_specs=pl.BlockSpec((BR, D_HID), lambda i: (i, 0)),
        out_shape=jax.ShapeDtypeStruct((NP, D_HID), jnp.float32),
    )(degP, h1)


def _post1(degP, S, g1, b1, W2):
    return pl.pallas_call(
        _post1_body,
        grid=(GRID,),
        in_specs=[
            _deg_spec(),
            pl.BlockSpec((NC, BR, D_HID), lambda i: (0, i, 0)),
            pl.BlockSpec((BR, D_HID), lambda i: (i, 0)),
            pl.BlockSpec((1, D_HID), lambda i: (0, 0)),
            pl.BlockSpec((D_HID, D_OUT), lambda i: (0, 0)),
        ],
        out_specs=pl.BlockSpec((BR, D_OUT), lambda i: (i, 0)),
        out_shape=jax.ShapeDtypeStruct((NP, D_OUT), jnp.float32),
    )(degP, S, g1, b1, W2)


def _post2(degP, T, g2, b2):
    return pl.pallas_call(
        _post2_body,
        grid=(GRID,),
        in_specs=[
            _deg_spec(),
            pl.BlockSpec((NC, BR, D_OUT), lambda i: (0, i, 0)),
            pl.BlockSpec((BR, D_OUT), lambda i: (i, 0)),
            pl.BlockSpec((1, D_OUT), lambda i: (0, 0)),
        ],
        out_specs=pl.BlockSpec((BR, D_OUT), lambda i: (i, 0)),
        out_shape=jax.ShapeDtypeStruct((NP, D_OUT), jnp.float32),
    )(degP, T, g2, b2)


def kernel(x, edge_index, W1, b1, W2, b2):
    ei3 = edge_index.astype(jnp.int32).reshape(2, NCHUNK_R, CW)
    padc = (N_NODES + (jnp.arange(PW_LAST * CW, dtype=jnp.int32)
                       % (NP - N_NODES))).reshape(PW_LAST, CW)
    x_p = jnp.pad(x, ((0, NP - N_NODES), (0, 0)))
    z1 = jnp.zeros((NP,), jnp.float32)
    z64 = jnp.zeros((NP, D_HID), jnp.float32)
    z16 = jnp.zeros((NP, D_OUT), jnp.float32)

    degP = _deg_kernel(ei3, padc, z1)
    h1 = _mm(x_p, W1)
    g1 = _scale(degP, h1)
    S = _agg64(g1, ei3, padc, z64)
    g2 = _post1(degP, S, g1, b1.reshape(1, -1), W2)
    T = _agg16(g2, ei3, padc, z16)
    o = _post2(degP, T, g2, b2.reshape(1, -1))
    return o[:N_NODES]


# speedup vs baseline: 1.1167x; 1.0102x over previous
import functools

import jax
import jax.numpy as jnp
from jax import lax
from jax.experimental import pallas as pl
from jax.experimental.pallas import tpu as pltpu
from jax.experimental.pallas import tpu_sc as plsc

N_NODES = 10000
NP = 10240
N_EDGES = 320000
CW = 128
CPW = 80
NC, NS = 2, 16
NW = NC * NS
NCHUNK = NW * CPW
E_PAD = NCHUNK * CW
D_IN, D_HID, D_OUT = 128, 64, 16
RPS = NP // NS

BPG = 1

NCHUNK_R = N_EDGES // CW
RW_LAST = NCHUNK_R - (NW - 1) * CPW
PW_LAST = CPW - RW_LAST

_sc_mesh = plsc.VectorSubcoreMesh(core_axis_name="c", subcore_axis_name="s")
_sc_params = pltpu.CompilerParams(use_tc_tiling_on_sc=False,
                                  needs_layout_passes=False)


def _hoist_idx(ei3_hbm, padc_hbm, idx_v, which, w):
    @pl.when(w < NW - 1)
    def _():
        pltpu.sync_copy(ei3_hbm.at[which, pl.ds(w * CPW, CPW)], idx_v)

    @pl.when(w == NW - 1)
    def _():
        pltpu.sync_copy(ei3_hbm.at[which, pl.ds((NW - 1) * CPW, RW_LAST)],
                        idx_v.at[pl.ds(0, RW_LAST)])
        pltpu.sync_copy(padc_hbm, idx_v.at[pl.ds(RW_LAST, PW_LAST)])


@functools.partial(
    pl.kernel,
    out_type=jax.ShapeDtypeStruct((NW, NP), jnp.float32),
    mesh=_sc_mesh,
    compiler_params=_sc_params,
    scratch_types=[
        pltpu.VMEM((CPW, CW), jnp.int32),
        pltpu.VMEM((NP,), jnp.float32),
    ],
)
def _deg_kernel(ei3_hbm, padc_hbm, zeros_hbm, out_hbm, cidx, hist):
    c = lax.axis_index("c")
    s = lax.axis_index("s")
    w = s * NC + c
    pltpu.sync_copy(zeros_hbm, hist)
    _hoist_idx(ei3_hbm, padc_hbm, cidx, 1, w)
    ones = jnp.full((16,), 1.0, jnp.float32)

    def body(j, carry):
        for k in range(CW // 16):
            idx = cidx[j, pl.ds(k * 16, 16)]
            plsc.addupdate_scatter(hist, [idx], ones)
        return carry

    lax.fori_loop(0, CPW, body, 0)
    pltpu.sync_copy(hist, out_hbm.at[w])


def _make_agg(D, G):
    NPASS = CPW // (G * BPG)
    @functools.partial(
        pl.kernel,
        out_type=jax.ShapeDtypeStruct((NC, NP, D), jnp.float32),
        mesh=_sc_mesh,
        compiler_params=_sc_params,
        scratch_types=[
            pltpu.VMEM((CPW, CW), jnp.int32),
            pltpu.VMEM((CPW, CW), jnp.int32),
            pltpu.VMEM((G * BPG * CW, D), jnp.float32),
            pltpu.VMEM_SHARED((NP, D), jnp.float32),
            [pltpu.SemaphoreType.DMA] * G,
            [pltpu.SemaphoreType.DMA] * G,
        ],
    )
    def agg(g_hbm, ei3_hbm, padc_hbm, zeros_hbm, out_hbm,
            ridx, cidx, gbuf, acc, gsems, ssems):
        c = lax.axis_index("c")
        s = lax.axis_index("s")
        w = s * NC + c
        pltpu.sync_copy(zeros_hbm.at[pl.ds(s * RPS, RPS)],
                        acc.at[pl.ds(s * RPS, RPS)])
        _hoist_idx(ei3_hbm, padc_hbm, ridx, 0, w)
        _hoist_idx(ei3_hbm, padc_hbm, cidx, 1, w)
        plsc.subcore_barrier()

        def gather(gi, j, k):
            pltpu.async_copy(g_hbm.at[ridx.at[k]],
                             gbuf.at[pl.ds((gi * BPG + j) * CW, CW)],
                             gsems[gi])

        def scatter(gi, j, k):
            pltpu.async_copy(gbuf.at[pl.ds((gi * BPG + j) * CW, CW)],
                             acc.at[cidx.at[k]], ssems[gi], add=True)

        def drain(sem, gi):
            pltpu.make_async_copy(
                zeros_hbm.at[pl.ds(0, BPG * CW)],
                gbuf.at[pl.ds(gi * BPG * CW, BPG * CW)], sem).wait()

        for gi in range(G):
            for j in range(BPG):
                gather(gi, j, gi * BPG + j)

        def body(t, carry):
            for gi in range(G):
                rp = G * (t - 1) + gi
                drain(gsems[gi], gi)
                for j in range(BPG):
                    scatter(gi, j, rp * BPG + j)
                drain(ssems[gi], gi)
                rc = G * t + gi
                for j in range(BPG):
                    gather(gi, j, rc * BPG + j)
            return carry

        lax.fori_loop(1, NPASS, body, 0)

        for gi in range(G):
            rp = G * (NPASS - 1) + gi
            drain(gsems[gi], gi)
            for j in range(BPG):
                scatter(gi, j, rp * BPG + j)
            drain(ssems[gi], gi)

        plsc.subcore_barrier()
        pltpu.sync_copy(acc.at[pl.ds(s * RPS, RPS)],
                        out_hbm.at[c, pl.ds(s * RPS, RPS)])

    return agg


_agg64 = _make_agg(D_HID, 8)
_agg16 = _make_agg(D_OUT, 8)


BR = 2048
GRID = NP // BR


def _dinv_of(deg_ref):
    i = pl.program_id(0)
    sl = deg_ref[:, pl.ds(i * BR, BR)]
    return lax.rsqrt(jnp.sum(sl, axis=0) + 1.0)[:, None]


def _mm_body(x_ref, w1_ref, h1_ref):
    h1_ref[...] = jnp.dot(x_ref[...], w1_ref[...],
                          preferred_element_type=jnp.float32)


def _scale_body(deg_ref, h_ref, g_ref):
    g_ref[...] = h_ref[...] * _dinv_of(deg_ref)


def _post1_body(deg_ref, s_ref, g1_ref, b1_ref, w2_ref, g2_ref):
    dinv = _dinv_of(deg_ref)
    out1 = jnp.maximum((s_ref[0] + s_ref[1] + g1_ref[...]) * dinv + b1_ref[...],
                       0.0)
    h2 = jnp.dot(out1, w2_ref[...], preferred_element_type=jnp.float32)
    g2_ref[...] = h2 * dinv


def _post2_body(deg_ref, t_ref, g2_ref, b2_ref, o_ref):
    dinv = _dinv_of(deg_ref)
    o = (t_ref[0] + t_ref[1] + g2_ref[...]) * dinv + b2_ref[...]
    m = jnp.max(o, axis=1, keepdims=True)
    o_ref[...] = o - (jnp.log(jnp.sum(jnp.exp(o - m), axis=1, keepdims=True)) + m)


def _deg_spec():
    return pl.BlockSpec((NW, NP), lambda i: (0, 0))


def _mm(x_p, W1):
    return pl.pallas_call(
        _mm_body,
        grid=(GRID,),
        in_specs=[
            pl.BlockSpec((BR, D_IN), lambda i: (i, 0)),
            pl.BlockSpec((D_IN, D_HID), lambda i: (0, 0)),
        ],
        out_specs=pl.BlockSpec((BR, D_HID), lambda i: (i, 0)),
        out_shape=jax.ShapeDtypeStruct((NP, D_HID), jnp.float32),
    )(x_p, W1)


def _scale(degP, h1):
    return pl.pallas_call(
        _scale_body,
        grid=(GRID,),
        in_specs=[
            _deg_spec(),
            pl.BlockSpec((BR, D_HID), lambda i: (i, 0)),
        ],
        out_specs=pl.BlockSpec((BR, D_HID), lambda i: (i, 0)),
        out_shape=jax.ShapeDtypeStruct((NP, D_HID), jnp.float32),
    )(degP, h1)


def _post1(degP, S, g1, b1, W2):
    return pl.pallas_call(
        _post1_body,
        grid=(GRID,),
        in_specs=[
            _deg_spec(),
            pl.BlockSpec((NC, BR, D_HID), lambda i: (0, i, 0)),
            pl.BlockSpec((BR, D_HID), lambda i: (i, 0)),
            pl.BlockSpec((1, D_HID), lambda i: (0, 0)),
            pl.BlockSpec((D_HID, D_OUT), lambda i: (0, 0)),
        ],
        out_specs=pl.BlockSpec((BR, D_OUT), lambda i: (i, 0)),
        out_shape=jax.ShapeDtypeStruct((NP, D_OUT), jnp.float32),
    )(degP, S, g1, b1, W2)


def _post2(degP, T, g2, b2):
    return pl.pallas_call(
        _post2_body,
        grid=(GRID,),
        in_specs=[
            _deg_spec(),
            pl.BlockSpec((NC, BR, D_OUT), lambda i: (0, i, 0)),
            pl.BlockSpec((BR, D_OUT), lambda i: (i, 0)),
            pl.BlockSpec((1, D_OUT), lambda i: (0, 0)),
        ],
        out_specs=pl.BlockSpec((BR, D_OUT), lambda i: (i, 0)),
        out_shape=jax.ShapeDtypeStruct((NP, D_OUT), jnp.float32),
    )(degP, T, g2, b2)


def kernel(x, edge_index, W1, b1, W2, b2):
    ei3 = edge_index.astype(jnp.int32).reshape(2, NCHUNK_R, CW)
    padc = (N_NODES + (jnp.arange(PW_LAST * CW, dtype=jnp.int32)
                       % (NP - N_NODES))).reshape(PW_LAST, CW)
    x_p = jnp.pad(x, ((0, NP - N_NODES), (0, 0)))
    z1 = jnp.zeros((NP,), jnp.float32)
    z64 = jnp.zeros((NP, D_HID), jnp.float32)
    z16 = jnp.zeros((NP, D_OUT), jnp.float32)

    degP = _deg_kernel(ei3, padc, z1)
    h1 = _mm(x_p, W1)
    g1 = _scale(degP, h1)
    S = _agg64(g1, ei3, padc, z64)
    g2 = _post1(degP, S, g1, b1.reshape(1, -1), W2)
    T = _agg16(g2, ei3, padc, z16)
    o = _post2(degP, T, g2, b2.reshape(1, -1))
    return o[:N_NODES]
